# Initial kernel scaffold; baseline (speedup 1.0000x reference)
import jax
import jax.numpy as jnp
from jax.experimental import pallas as pl


def kernel(features, ins_indices_batch, ins_ids, weight, bias):
    raise NotImplementedError("write your pallas kernel here")


# speedup vs baseline: 3.4758x; 3.4758x over previous
import jax
import jax.numpy as jnp
from jax import lax
from jax.experimental import pallas as pl
from jax.experimental.pallas import tpu as pltpu

_N = 32768
_C = 256
_G = 32
_CPG = _C // _G
_NI = 64
_EPS = 1e-5
_BLK = 1024
_NBLK = _N // _BLK
_HI = lax.Precision.HIGHEST


def _stats_body(seg_ref, x_ref, s1_ref, s2_ref, cnt_ref):
    i = pl.program_id(0)

    @pl.when(i == 0)
    def _init():
        s1_ref[...] = jnp.zeros_like(s1_ref)
        s2_ref[...] = jnp.zeros_like(s2_ref)
        cnt_ref[...] = jnp.zeros_like(cnt_ref)

    x = x_ref[...]
    seg = seg_ref[0]
    eg = (
        lax.broadcasted_iota(jnp.int32, (_C, _G), 0) // _CPG
        == lax.broadcasted_iota(jnp.int32, (_C, _G), 1)
    ).astype(jnp.float32)
    ts1 = jnp.dot(x, eg, precision=_HI)
    ts2 = jnp.dot(x * x, eg, precision=_HI)
    onehot = (
        lax.broadcasted_iota(jnp.int32, (_NI, _BLK), 0) == seg
    ).astype(jnp.float32)
    s1_ref[...] += jnp.dot(onehot, ts1, precision=_HI)
    s2_ref[...] += jnp.dot(onehot, ts2, precision=_HI)
    cnt_ref[...] += jnp.broadcast_to(
        jnp.sum(onehot, axis=1, keepdims=True), (_NI, _G)
    )


def _norm_body(seg_ref, s1_ref, s2_ref, cnt_ref, x_ref, w_ref, b_ref, o_ref):
    cnt = jnp.maximum(cnt_ref[...] * float(_CPG), 1.0)
    mean = s1_ref[...] / cnt
    var = s2_ref[...] / cnt - mean * mean
    inv = lax.rsqrt(var + _EPS)
    rg = (
        lax.broadcasted_iota(jnp.int32, (_G, _C), 0)
        == lax.broadcasted_iota(jnp.int32, (_G, _C), 1) // _CPG
    ).astype(jnp.float32)
    inv_c = jnp.dot(inv, rg, precision=_HI)
    mean_c = jnp.dot(mean, rg, precision=_HI)
    scale = inv_c * w_ref[...]
    shift = b_ref[...] - mean_c * scale
    segc = seg_ref[0]
    onehot = (
        segc == lax.broadcasted_iota(jnp.int32, (_BLK, _NI), 1)
    ).astype(jnp.float32)
    sc_tok = jnp.dot(onehot, scale, precision=_HI)
    sh_tok = jnp.dot(onehot, shift, precision=_HI)
    o_ref[...] = x_ref[...] * sc_tok + sh_tok


def kernel(features, ins_indices_batch, ins_ids, weight, bias):
    del ins_ids
    seg = ins_indices_batch.astype(jnp.int32)
    seg_row = seg.reshape(_NBLK, 1, _BLK)
    seg_col = seg.reshape(_NBLK, _BLK, 1)
    w2 = weight.reshape(1, _C)
    b2 = bias.reshape(1, _C)

    s1, s2, cnt = pl.pallas_call(
        _stats_body,
        grid=(_NBLK,),
        in_specs=[
            pl.BlockSpec((1, 1, _BLK), lambda i: (i, 0, 0)),
            pl.BlockSpec((_BLK, _C), lambda i: (i, 0)),
        ],
        out_specs=[
            pl.BlockSpec((_NI, _G), lambda i: (0, 0)),
            pl.BlockSpec((_NI, _G), lambda i: (0, 0)),
            pl.BlockSpec((_NI, _G), lambda i: (0, 0)),
        ],
        out_shape=[
            jax.ShapeDtypeStruct((_NI, _G), jnp.float32),
            jax.ShapeDtypeStruct((_NI, _G), jnp.float32),
            jax.ShapeDtypeStruct((_NI, _G), jnp.float32),
        ],
    )(seg_row, features)

    out = pl.pallas_call(
        _norm_body,
        grid=(_NBLK,),
        in_specs=[
            pl.BlockSpec((1, _BLK, 1), lambda i: (i, 0, 0)),
            pl.BlockSpec((_NI, _G), lambda i: (0, 0)),
            pl.BlockSpec((_NI, _G), lambda i: (0, 0)),
            pl.BlockSpec((_NI, _G), lambda i: (0, 0)),
            pl.BlockSpec((_BLK, _C), lambda i: (i, 0)),
            pl.BlockSpec((1, _C), lambda i: (0, 0)),
            pl.BlockSpec((1, _C), lambda i: (0, 0)),
        ],
        out_specs=pl.BlockSpec((_BLK, _C), lambda i: (i, 0)),
        out_shape=jax.ShapeDtypeStruct((_N, _C), jnp.float32),
    )(seg_col, s1, s2, cnt, features, w2, b2)
    return out


# speedup vs baseline: 5.5739x; 1.6036x over previous
import jax
import jax.numpy as jnp
from jax import lax
from jax.experimental import pallas as pl
from jax.experimental.pallas import tpu as pltpu

_N = 32768
_C = 256
_G = 32
_CPG = _C // _G
_NI = 64
_EPS = 1e-5
_BLK = 1024
_NBLK = _N // _BLK


def _hilo(v):
    hi = v.astype(jnp.bfloat16)
    lo = (v - hi.astype(jnp.float32)).astype(jnp.bfloat16)
    return hi, lo


def _stats_body(seg_ref, x_ref, s12_ref, cnt_ref):
    i = pl.program_id(0)

    @pl.when(i == 0)
    def _init():
        s12_ref[...] = jnp.zeros_like(s12_ref)
        cnt_ref[...] = jnp.zeros_like(cnt_ref)

    x = x_ref[...]
    seg = seg_ref[0]
    onehot = (
        lax.broadcasted_iota(jnp.int32, (_NI, _BLK), 0) == seg
    ).astype(jnp.bfloat16)
    x2 = jnp.concatenate([x, x * x], axis=1)
    hi, lo = _hilo(x2)
    acc = jnp.dot(onehot, hi, preferred_element_type=jnp.float32)
    acc += jnp.dot(onehot, lo, preferred_element_type=jnp.float32)
    s12_ref[...] += acc
    cnt_ref[...] += jnp.broadcast_to(
        jnp.sum(onehot.astype(jnp.float32), axis=1, keepdims=True), (_NI, _G)
    )


def _norm_body(seg_ref, s12_ref, cnt_ref, x_ref, w_ref, b_ref, o_ref):
    s12 = s12_ref[...]
    eg = (
        lax.broadcasted_iota(jnp.int32, (2 * _C, 2 * _G), 0) // _CPG
        == lax.broadcasted_iota(jnp.int32, (2 * _C, 2 * _G), 1)
    ).astype(jnp.float32)
    sg = jnp.dot(s12, eg, precision=lax.Precision.HIGHEST)
    s1 = sg[:, :_G]
    s2 = sg[:, _G:]
    cnt = jnp.maximum(cnt_ref[...] * float(_CPG), 1.0)
    mean = s1 / cnt
    var = s2 / cnt - mean * mean
    inv = lax.rsqrt(var + _EPS)
    rg = (
        lax.broadcasted_iota(jnp.int32, (_G, _C), 0)
        == lax.broadcasted_iota(jnp.int32, (_G, _C), 1) // _CPG
    ).astype(jnp.float32)
    inv_c = jnp.dot(inv, rg, precision=lax.Precision.HIGHEST)
    mean_c = jnp.dot(mean, rg, precision=lax.Precision.HIGHEST)
    scale = inv_c * w_ref[...]
    shift = b_ref[...] - mean_c * scale
    sc_hi, sc_lo = _hilo(scale)
    sh_hi, sh_lo = _hilo(shift)
    tab = jnp.concatenate([sc_hi, sc_lo, sh_hi, sh_lo], axis=1)
    segc = seg_ref[0]
    onehot = (
        segc == lax.broadcasted_iota(jnp.int32, (_BLK, _NI), 1)
    ).astype(jnp.bfloat16)
    prod = jnp.dot(onehot, tab, preferred_element_type=jnp.float32)
    sc_tok = prod[:, :_C] + prod[:, _C : 2 * _C]
    sh_tok = prod[:, 2 * _C : 3 * _C] + prod[:, 3 * _C :]
    o_ref[...] = x_ref[...] * sc_tok + sh_tok


def kernel(features, ins_indices_batch, ins_ids, weight, bias):
    del ins_ids
    seg = ins_indices_batch.astype(jnp.int32)
    seg_row = seg.reshape(_NBLK, 1, _BLK)
    seg_col = seg.reshape(_NBLK, _BLK, 1)
    w2 = weight.reshape(1, _C)
    b2 = bias.reshape(1, _C)

    s12, cnt = pl.pallas_call(
        _stats_body,
        grid=(_NBLK,),
        in_specs=[
            pl.BlockSpec((1, 1, _BLK), lambda i: (i, 0, 0)),
            pl.BlockSpec((_BLK, _C), lambda i: (i, 0)),
        ],
        out_specs=[
            pl.BlockSpec((_NI, 2 * _C), lambda i: (0, 0)),
            pl.BlockSpec((_NI, _G), lambda i: (0, 0)),
        ],
        out_shape=[
            jax.ShapeDtypeStruct((_NI, 2 * _C), jnp.float32),
            jax.ShapeDtypeStruct((_NI, _G), jnp.float32),
        ],
    )(seg_row, features)

    out = pl.pallas_call(
        _norm_body,
        grid=(_NBLK,),
        in_specs=[
            pl.BlockSpec((1, _BLK, 1), lambda i: (i, 0, 0)),
            pl.BlockSpec((_NI, 2 * _C), lambda i: (0, 0)),
            pl.BlockSpec((_NI, _G), lambda i: (0, 0)),
            pl.BlockSpec((_BLK, _C), lambda i: (i, 0)),
            pl.BlockSpec((1, _C), lambda i: (0, 0)),
            pl.BlockSpec((1, _C), lambda i: (0, 0)),
        ],
        out_specs=pl.BlockSpec((_BLK, _C), lambda i: (i, 0)),
        out_shape=jax.ShapeDtypeStruct((_N, _C), jnp.float32),
    )(seg_col, s12, cnt, features, w2, b2)
    return out


# speedup vs baseline: 6.0688x; 1.0888x over previous
import jax
import jax.numpy as jnp
from jax import lax
from jax.experimental import pallas as pl
from jax.experimental.pallas import tpu as pltpu

_N = 32768
_C = 256
_G = 32
_CPG = _C // _G
_NI = 64
_EPS = 1e-5
_BLK = 1024
_NBLK = _N // _BLK
_HI = lax.Precision.HIGHEST


def _hilo(v):
    hi = v.astype(jnp.bfloat16)
    lo = (v - hi.astype(jnp.float32)).astype(jnp.bfloat16)
    return hi, lo


def _stats_body(seg_ref, x_ref, w_ref, b_ref, tab_ref, s12_acc, cnt_acc):
    i = pl.program_id(0)

    @pl.when(i == 0)
    def _init():
        s12_acc[...] = jnp.zeros_like(s12_acc)
        cnt_acc[...] = jnp.zeros_like(cnt_acc)

    x = x_ref[...]
    seg = seg_ref[0]
    onehot = (
        lax.broadcasted_iota(jnp.int32, (_NI, _BLK), 0) == seg
    ).astype(jnp.bfloat16)
    x2 = jnp.concatenate([x, x * x], axis=1)
    hi, lo = _hilo(x2)
    acc = jnp.dot(onehot, hi, preferred_element_type=jnp.float32)
    acc += jnp.dot(onehot, lo, preferred_element_type=jnp.float32)
    s12_acc[...] += acc
    cnt_acc[...] += jnp.sum(onehot.astype(jnp.float32), axis=1, keepdims=True)

    @pl.when(i == _NBLK - 1)
    def _epilogue():
        s12 = s12_acc[...]
        eg = (
            lax.broadcasted_iota(jnp.int32, (2 * _C, 2 * _G), 0) // _CPG
            == lax.broadcasted_iota(jnp.int32, (2 * _C, 2 * _G), 1)
        ).astype(jnp.float32)
        sg = jnp.dot(s12, eg, precision=_HI)
        cnt = jnp.maximum(cnt_acc[...] * float(_CPG), 1.0)
        mean = sg[:, :_G] / cnt
        var = sg[:, _G:] / cnt - mean * mean
        inv = lax.rsqrt(var + _EPS)
        rg = (
            lax.broadcasted_iota(jnp.int32, (_G, _C), 0)
            == lax.broadcasted_iota(jnp.int32, (_G, _C), 1) // _CPG
        ).astype(jnp.float32)
        inv_c = jnp.dot(inv, rg, precision=_HI)
        mean_c = jnp.dot(mean, rg, precision=_HI)
        scale = inv_c * w_ref[...]
        shift = b_ref[...] - mean_c * scale
        sc_hi, sc_lo = _hilo(scale)
        sh_hi, sh_lo = _hilo(shift)
        tab_ref[...] = jnp.concatenate([sc_hi, sc_lo, sh_hi, sh_lo], axis=1)


def _norm_body(seg_ref, tab_ref, x_ref, o_ref):
    segc = seg_ref[0]
    onehot = (
        segc == lax.broadcasted_iota(jnp.int32, (_BLK, _NI), 1)
    ).astype(jnp.bfloat16)
    prod = jnp.dot(onehot, tab_ref[...], preferred_element_type=jnp.float32)
    sc_tok = prod[:, :_C] + prod[:, _C : 2 * _C]
    sh_tok = prod[:, 2 * _C : 3 * _C] + prod[:, 3 * _C :]
    o_ref[...] = x_ref[...] * sc_tok + sh_tok


def kernel(features, ins_indices_batch, ins_ids, weight, bias):
    del ins_ids
    seg = ins_indices_batch.astype(jnp.int32)
    seg_row = seg.reshape(_NBLK, 1, _BLK)
    seg_col = seg.reshape(_NBLK, _BLK, 1)
    w2 = weight.reshape(1, _C)
    b2 = bias.reshape(1, _C)

    tab = pl.pallas_call(
        _stats_body,
        grid=(_NBLK,),
        in_specs=[
            pl.BlockSpec((1, 1, _BLK), lambda i: (i, 0, 0)),
            pl.BlockSpec((_BLK, _C), lambda i: (i, 0)),
            pl.BlockSpec((1, _C), lambda i: (0, 0)),
            pl.BlockSpec((1, _C), lambda i: (0, 0)),
        ],
        out_specs=pl.BlockSpec((_NI, 4 * _C), lambda i: (0, 0)),
        out_shape=jax.ShapeDtypeStruct((_NI, 4 * _C), jnp.bfloat16),
        scratch_shapes=[
            pltpu.VMEM((_NI, 2 * _C), jnp.float32),
            pltpu.VMEM((_NI, 1), jnp.float32),
        ],
    )(seg_row, features, w2, b2)

    out = pl.pallas_call(
        _norm_body,
        grid=(_NBLK,),
        in_specs=[
            pl.BlockSpec((1, _BLK, 1), lambda i: (i, 0, 0)),
            pl.BlockSpec((_NI, 4 * _C), lambda i: (0, 0)),
            pl.BlockSpec((_BLK, _C), lambda i: (i, 0)),
        ],
        out_specs=pl.BlockSpec((_BLK, _C), lambda i: (i, 0)),
        out_shape=jax.ShapeDtypeStruct((_N, _C), jnp.float32),
    )(seg_col, tab, features)
    return out


# speedup vs baseline: 7.9187x; 1.3048x over previous
import jax
import jax.numpy as jnp
from jax import lax
from jax.experimental import pallas as pl
from jax.experimental.pallas import tpu as pltpu

_N = 32768
_C = 256
_G = 32
_CPG = _C // _G
_NI = 64
_EPS = 1e-5
_BLK = 2048
_NBLK = _N // _BLK
_HI = lax.Precision.HIGHEST


def _hilo(v):
    hi = v.astype(jnp.bfloat16)
    lo = (v - hi.astype(jnp.float32)).astype(jnp.bfloat16)
    return hi, lo


def _stats_body(seg_ref, x_ref, w_ref, b_ref, tab_ref, s1_acc, s2_acc, cnt_acc):
    i = pl.program_id(0)

    @pl.when(i == 0)
    def _init():
        s1_acc[...] = jnp.zeros_like(s1_acc)
        s2_acc[...] = jnp.zeros_like(s2_acc)
        cnt_acc[...] = jnp.zeros_like(cnt_acc)

    xb = x_ref[...].astype(jnp.bfloat16)
    seg = seg_ref[0]
    onehot = (
        lax.broadcasted_iota(jnp.int32, (_NI, _BLK), 0) == seg
    ).astype(jnp.bfloat16)
    s1_acc[...] += jnp.dot(onehot, xb, preferred_element_type=jnp.float32)
    s2_acc[...] += jnp.dot(onehot, xb * xb, preferred_element_type=jnp.float32)
    cnt_acc[...] += jnp.sum(onehot.astype(jnp.float32), axis=1, keepdims=True)

    @pl.when(i == _NBLK - 1)
    def _epilogue():
        s12 = jnp.concatenate([s1_acc[...], s2_acc[...]], axis=1)
        eg = (
            lax.broadcasted_iota(jnp.int32, (2 * _C, 2 * _G), 0) // _CPG
            == lax.broadcasted_iota(jnp.int32, (2 * _C, 2 * _G), 1)
        ).astype(jnp.float32)
        sg = jnp.dot(s12, eg, precision=_HI)
        cnt = jnp.maximum(cnt_acc[...] * float(_CPG), 1.0)
        mean = sg[:, :_G] / cnt
        var = sg[:, _G:] / cnt - mean * mean
        inv = lax.rsqrt(var + _EPS)
        rg = (
            lax.broadcasted_iota(jnp.int32, (_G, _C), 0)
            == lax.broadcasted_iota(jnp.int32, (_G, _C), 1) // _CPG
        ).astype(jnp.float32)
        inv_c = jnp.dot(inv, rg, precision=_HI)
        mean_c = jnp.dot(mean, rg, precision=_HI)
        scale = inv_c * w_ref[...]
        shift = b_ref[...] - mean_c * scale
        sc_hi, sc_lo = _hilo(scale)
        sh_hi, sh_lo = _hilo(shift)
        tab_ref[...] = jnp.concatenate([sc_hi, sc_lo, sh_hi, sh_lo], axis=1)


def _norm_body(seg_ref, tab_ref, x_ref, o_ref):
    segc = seg_ref[0]
    onehot = (
        segc == lax.broadcasted_iota(jnp.int32, (_BLK, _NI), 1)
    ).astype(jnp.bfloat16)
    prod = jnp.dot(onehot, tab_ref[...], preferred_element_type=jnp.float32)
    sc_tok = prod[:, :_C] + prod[:, _C : 2 * _C]
    sh_tok = prod[:, 2 * _C : 3 * _C] + prod[:, 3 * _C :]
    o_ref[...] = x_ref[...] * sc_tok + sh_tok


def kernel(features, ins_indices_batch, ins_ids, weight, bias):
    del ins_ids
    seg = ins_indices_batch.astype(jnp.int32)
    seg_row = seg.reshape(_NBLK, 1, _BLK)
    seg_col = seg.reshape(_NBLK, _BLK, 1)
    w2 = weight.reshape(1, _C)
    b2 = bias.reshape(1, _C)

    tab = pl.pallas_call(
        _stats_body,
        grid=(_NBLK,),
        in_specs=[
            pl.BlockSpec((1, 1, _BLK), lambda i: (i, 0, 0)),
            pl.BlockSpec((_BLK, _C), lambda i: (i, 0)),
            pl.BlockSpec((1, _C), lambda i: (0, 0)),
            pl.BlockSpec((1, _C), lambda i: (0, 0)),
        ],
        out_specs=pl.BlockSpec((_NI, 4 * _C), lambda i: (0, 0)),
        out_shape=jax.ShapeDtypeStruct((_NI, 4 * _C), jnp.bfloat16),
        scratch_shapes=[
            pltpu.VMEM((_NI, _C), jnp.float32),
            pltpu.VMEM((_NI, _C), jnp.float32),
            pltpu.VMEM((_NI, 1), jnp.float32),
        ],
    )(seg_row, features, w2, b2)

    out = pl.pallas_call(
        _norm_body,
        grid=(_NBLK,),
        in_specs=[
            pl.BlockSpec((1, _BLK, 1), lambda i: (i, 0, 0)),
            pl.BlockSpec((_NI, 4 * _C), lambda i: (0, 0)),
            pl.BlockSpec((_BLK, _C), lambda i: (i, 0)),
        ],
        out_specs=pl.BlockSpec((_BLK, _C), lambda i: (i, 0)),
        out_shape=jax.ShapeDtypeStruct((_N, _C), jnp.float32),
    )(seg_col, tab, features)
    return out


# speedup vs baseline: 8.4420x; 1.0661x over previous
import jax
import jax.numpy as jnp
from jax import lax
from jax.experimental import pallas as pl
from jax.experimental.pallas import tpu as pltpu

_N = 32768
_C = 256
_G = 32
_CPG = _C // _G
_NI = 64
_EPS = 1e-5
_BLK = 2048
_NBLK = _N // _BLK
_HI = lax.Precision.HIGHEST


def _hilo(v):
    hi = v.astype(jnp.bfloat16)
    lo = (v - hi.astype(jnp.float32)).astype(jnp.bfloat16)
    return hi, lo


def _body(seg_row_ref, seg_col_ref, x_ref, w_ref, b_ref, o_ref,
          s1_acc, s2_acc, cnt_acc, tab, xsave):
    p = pl.program_id(0)
    i = pl.program_id(1)

    @pl.when(p == 0)
    def _stats():
        @pl.when(i == 0)
        def _init():
            s1_acc[...] = jnp.zeros_like(s1_acc)
            s2_acc[...] = jnp.zeros_like(s2_acc)
            cnt_acc[...] = jnp.zeros_like(cnt_acc)

        x = x_ref[...]
        xsave[pl.ds(i * _BLK, _BLK), :] = x
        xb = x.astype(jnp.bfloat16)
        seg = seg_row_ref[0]
        onehot = (
            lax.broadcasted_iota(jnp.int32, (_NI, _BLK), 0) == seg
        ).astype(jnp.bfloat16)
        s1_acc[...] += jnp.dot(onehot, xb, preferred_element_type=jnp.float32)
        s2_acc[...] += jnp.dot(onehot, xb * xb, preferred_element_type=jnp.float32)
        cnt_acc[...] += jnp.sum(
            onehot.astype(jnp.float32), axis=1, keepdims=True
        )

        @pl.when(i == _NBLK - 1)
        def _epilogue():
            s12 = jnp.concatenate([s1_acc[...], s2_acc[...]], axis=1)
            eg = (
                lax.broadcasted_iota(jnp.int32, (2 * _C, 2 * _G), 0) // _CPG
                == lax.broadcasted_iota(jnp.int32, (2 * _C, 2 * _G), 1)
            ).astype(jnp.float32)
            sg = jnp.dot(s12, eg, precision=_HI)
            cnt = jnp.maximum(cnt_acc[...] * float(_CPG), 1.0)
            mean = sg[:, :_G] / cnt
            var = sg[:, _G:] / cnt - mean * mean
            inv = lax.rsqrt(var + _EPS)
            rg = (
                lax.broadcasted_iota(jnp.int32, (_G, _C), 0)
                == lax.broadcasted_iota(jnp.int32, (_G, _C), 1) // _CPG
            ).astype(jnp.float32)
            inv_c = jnp.dot(inv, rg, precision=_HI)
            mean_c = jnp.dot(mean, rg, precision=_HI)
            scale = inv_c * w_ref[...]
            shift = b_ref[...] - mean_c * scale
            sc_hi, sc_lo = _hilo(scale)
            sh_hi, sh_lo = _hilo(shift)
            tab[...] = jnp.concatenate([sc_hi, sc_lo, sh_hi, sh_lo], axis=1)

    @pl.when(p == 1)
    def _norm():
        segc = seg_col_ref[0]
        onehot = (
            segc == lax.broadcasted_iota(jnp.int32, (_BLK, _NI), 1)
        ).astype(jnp.bfloat16)
        t = tab[...]
        sc_tok = jnp.dot(
            onehot, t[:, :_C], preferred_element_type=jnp.float32
        ) + jnp.dot(onehot, t[:, _C : 2 * _C], preferred_element_type=jnp.float32)
        sh_tok = jnp.dot(
            onehot, t[:, 2 * _C : 3 * _C], preferred_element_type=jnp.float32
        ) + jnp.dot(onehot, t[:, 3 * _C :], preferred_element_type=jnp.float32)
        o_ref[...] = xsave[pl.ds(i * _BLK, _BLK), :] * sc_tok + sh_tok


def kernel(features, ins_indices_batch, ins_ids, weight, bias):
    del ins_ids
    seg = ins_indices_batch.astype(jnp.int32)
    seg_row = seg.reshape(_NBLK, 1, _BLK)
    seg_col = seg.reshape(_NBLK, _BLK, 1)
    w2 = weight.reshape(1, _C)
    b2 = bias.reshape(1, _C)

    out = pl.pallas_call(
        _body,
        grid=(2, _NBLK),
        in_specs=[
            pl.BlockSpec((1, 1, _BLK), lambda p, i: (i, 0, 0)),
            pl.BlockSpec((1, _BLK, 1), lambda p, i: (i, 0, 0)),
            pl.BlockSpec((_BLK, _C), lambda p, i: ((1 - p) * i, 0)),
            pl.BlockSpec((1, _C), lambda p, i: (0, 0)),
            pl.BlockSpec((1, _C), lambda p, i: (0, 0)),
        ],
        out_specs=pl.BlockSpec((_BLK, _C), lambda p, i: (p * i, 0)),
        out_shape=jax.ShapeDtypeStruct((_N, _C), jnp.float32),
        scratch_shapes=[
            pltpu.VMEM((_NI, _C), jnp.float32),
            pltpu.VMEM((_NI, _C), jnp.float32),
            pltpu.VMEM((_NI, 1), jnp.float32),
            pltpu.VMEM((_NI, 4 * _C), jnp.bfloat16),
            pltpu.VMEM((_N, _C), jnp.float32),
        ],
    )(seg_row, seg_col, features, w2, b2)
    return out


# speedup vs baseline: 8.7306x; 1.0342x over previous
import jax
import jax.numpy as jnp
from jax import lax
from jax.experimental import pallas as pl
from jax.experimental.pallas import tpu as pltpu

_N = 32768
_C = 256
_G = 32
_CPG = _C // _G
_NI = 64
_EPS = 1e-5
_BLK = 2048
_NBLK = _N // _BLK
_HI = lax.Precision.HIGHEST


def _hilo(v):
    hi = v.astype(jnp.bfloat16)
    lo = (v - hi.astype(jnp.float32)).astype(jnp.bfloat16)
    return hi, lo


def _body(seg_row_ref, seg_col_ref, x_ref, w_ref, b_ref, o_ref,
          s1_acc, s2_acc, cnt_acc, tab, xsave):
    p = pl.program_id(0)
    i = pl.program_id(1)

    @pl.when(p == 0)
    def _stats():
        @pl.when(i == 0)
        def _init():
            s1_acc[...] = jnp.zeros_like(s1_acc)
            s2_acc[...] = jnp.zeros_like(s2_acc)
            cnt_acc[...] = jnp.zeros_like(cnt_acc)

        x = x_ref[...]
        xsave[pl.ds(i * _BLK, _BLK), :] = x
        xb = x.astype(jnp.bfloat16)
        seg = seg_row_ref[0]
        onehot = (
            lax.broadcasted_iota(jnp.int32, (_NI, _BLK), 0) == seg
        ).astype(jnp.bfloat16)
        s1_acc[...] += jnp.dot(onehot, xb, preferred_element_type=jnp.float32)
        s2_acc[...] += jnp.dot(onehot, xb * xb, preferred_element_type=jnp.float32)
        cnt_acc[...] += jnp.sum(
            onehot.astype(jnp.float32), axis=1, keepdims=True
        )

        @pl.when(i == _NBLK - 1)
        def _epilogue():
            s12 = jnp.concatenate([s1_acc[...], s2_acc[...]], axis=1)
            eg = (
                lax.broadcasted_iota(jnp.int32, (2 * _C, 2 * _G), 0) // _CPG
                == lax.broadcasted_iota(jnp.int32, (2 * _C, 2 * _G), 1)
            ).astype(jnp.float32)
            sg = jnp.dot(s12, eg, precision=_HI)
            cnt = jnp.maximum(cnt_acc[...] * float(_CPG), 1.0)
            mean = sg[:, :_G] / cnt
            var = sg[:, _G:] / cnt - mean * mean
            inv = lax.rsqrt(var + _EPS)
            rg = (
                lax.broadcasted_iota(jnp.int32, (_G, _C), 0)
                == lax.broadcasted_iota(jnp.int32, (_G, _C), 1) // _CPG
            ).astype(jnp.float32)
            inv_c = jnp.dot(inv, rg, precision=_HI)
            mean_c = jnp.dot(mean, rg, precision=_HI)
            scale = inv_c * w_ref[...]
            shift = b_ref[...] - mean_c * scale
            sc_hi, sc_lo = _hilo(scale)
            sh = shift.astype(jnp.bfloat16)
            tab[...] = jnp.concatenate([sc_hi, sc_lo, sh], axis=1)

    @pl.when(p == 1)
    def _norm():
        segc = seg_col_ref[0]
        onehot = (
            segc == lax.broadcasted_iota(jnp.int32, (_BLK, _NI), 1)
        ).astype(jnp.bfloat16)
        t = tab[...]
        sc_tok = jnp.dot(
            onehot, t[:, :_C], preferred_element_type=jnp.float32
        ) + jnp.dot(onehot, t[:, _C : 2 * _C], preferred_element_type=jnp.float32)
        sh_tok = jnp.dot(
            onehot, t[:, 2 * _C : 3 * _C], preferred_element_type=jnp.float32
        )
        o_ref[...] = xsave[pl.ds(i * _BLK, _BLK), :] * sc_tok + sh_tok


def kernel(features, ins_indices_batch, ins_ids, weight, bias):
    del ins_ids
    seg = ins_indices_batch.astype(jnp.int32)
    seg_row = seg.reshape(_NBLK, 1, _BLK)
    seg_col = seg.reshape(_NBLK, _BLK, 1)
    w2 = weight.reshape(1, _C)
    b2 = bias.reshape(1, _C)

    out = pl.pallas_call(
        _body,
        grid=(2, _NBLK),
        in_specs=[
            pl.BlockSpec((1, 1, _BLK), lambda p, i: (i, 0, 0)),
            pl.BlockSpec((1, _BLK, 1), lambda p, i: (i, 0, 0)),
            pl.BlockSpec((_BLK, _C), lambda p, i: ((1 - p) * i, 0)),
            pl.BlockSpec((1, _C), lambda p, i: (0, 0)),
            pl.BlockSpec((1, _C), lambda p, i: (0, 0)),
        ],
        out_specs=pl.BlockSpec((_BLK, _C), lambda p, i: (p * i, 0)),
        out_shape=jax.ShapeDtypeStruct((_N, _C), jnp.float32),
        scratch_shapes=[
            pltpu.VMEM((_NI, _C), jnp.float32),
            pltpu.VMEM((_NI, _C), jnp.float32),
            pltpu.VMEM((_NI, 1), jnp.float32),
            pltpu.VMEM((_NI, 3 * _C), jnp.bfloat16),
            pltpu.VMEM((_N, _C), jnp.float32),
        ],
    )(seg_row, seg_col, features, w2, b2)
    return out


# speedup vs baseline: 10.0861x; 1.1552x over previous
import jax
import jax.numpy as jnp
from jax import lax
from jax.experimental import pallas as pl
from jax.experimental.pallas import tpu as pltpu

_N = 32768
_C = 256
_G = 32
_CPG = _C // _G
_NI = 64
_EPS = 1e-5
_BLK = 4096
_NBLK = _N // _BLK
_HI = lax.Precision.HIGHEST


def _hilo(v):
    hi = v.astype(jnp.bfloat16)
    lo = (v - hi.astype(jnp.float32)).astype(jnp.bfloat16)
    return hi, lo


def _body(seg_row_ref, seg_col_ref, x_ref, w_ref, b_ref, o_ref,
          s1_acc, s2_acc, cnt_acc, tab, xsave):
    p = pl.program_id(0)
    i = pl.program_id(1)

    @pl.when(p == 0)
    def _stats():
        @pl.when(i == 0)
        def _init():
            s1_acc[...] = jnp.zeros_like(s1_acc)
            s2_acc[...] = jnp.zeros_like(s2_acc)
            cnt_acc[...] = jnp.zeros_like(cnt_acc)

        x = x_ref[...]
        xsave[pl.ds(i * _BLK, _BLK), :] = x
        xb = x.astype(jnp.bfloat16)
        seg = seg_row_ref[0]
        onehot = (
            lax.broadcasted_iota(jnp.int32, (_NI, _BLK), 0) == seg
        ).astype(jnp.bfloat16)
        s1_acc[...] += jnp.dot(onehot, xb, preferred_element_type=jnp.float32)
        s2_acc[...] += jnp.dot(onehot, xb * xb, preferred_element_type=jnp.float32)
        cnt_acc[...] += jnp.sum(
            onehot.astype(jnp.float32), axis=1, keepdims=True
        )

        @pl.when(i == _NBLK - 1)
        def _epilogue():
            s12 = jnp.concatenate([s1_acc[...], s2_acc[...]], axis=1)
            eg = (
                lax.broadcasted_iota(jnp.int32, (2 * _C, 2 * _G), 0) // _CPG
                == lax.broadcasted_iota(jnp.int32, (2 * _C, 2 * _G), 1)
            ).astype(jnp.float32)
            sg = jnp.dot(s12, eg, precision=_HI)
            cnt = jnp.maximum(cnt_acc[...] * float(_CPG), 1.0)
            mean = sg[:, :_G] / cnt
            var = sg[:, _G:] / cnt - mean * mean
            inv = lax.rsqrt(var + _EPS)
            rg = (
                lax.broadcasted_iota(jnp.int32, (_G, _C), 0)
                == lax.broadcasted_iota(jnp.int32, (_G, _C), 1) // _CPG
            ).astype(jnp.float32)
            inv_c = jnp.dot(inv, rg, precision=_HI)
            mean_c = jnp.dot(mean, rg, precision=_HI)
            scale = inv_c * w_ref[...]
            shift = b_ref[...] - mean_c * scale
            sc_hi, sc_lo = _hilo(scale)
            sh = shift.astype(jnp.bfloat16)
            tab[...] = jnp.concatenate([sc_hi, sc_lo, sh], axis=1)

    @pl.when(p == 1)
    def _norm():
        segc = seg_col_ref[0]
        onehot = (
            segc == lax.broadcasted_iota(jnp.int32, (_BLK, _NI), 1)
        ).astype(jnp.bfloat16)
        t = tab[...]
        sc_tok = jnp.dot(
            onehot, t[:, :_C], preferred_element_type=jnp.float32
        ) + jnp.dot(onehot, t[:, _C : 2 * _C], preferred_element_type=jnp.float32)
        sh_tok = jnp.dot(
            onehot, t[:, 2 * _C : 3 * _C], preferred_element_type=jnp.float32
        )
        o_ref[...] = xsave[pl.ds(i * _BLK, _BLK), :] * sc_tok + sh_tok


def kernel(features, ins_indices_batch, ins_ids, weight, bias):
    del ins_ids
    seg = ins_indices_batch.astype(jnp.int32)
    seg_row = seg.reshape(_NBLK, 1, _BLK)
    seg_col = seg.reshape(_NBLK, _BLK, 1)
    w2 = weight.reshape(1, _C)
    b2 = bias.reshape(1, _C)

    out = pl.pallas_call(
        _body,
        grid=(2, _NBLK),
        in_specs=[
            pl.BlockSpec((1, 1, _BLK), lambda p, i: (i, 0, 0)),
            pl.BlockSpec((1, _BLK, 1), lambda p, i: (i, 0, 0)),
            pl.BlockSpec((_BLK, _C), lambda p, i: ((1 - p) * i, 0)),
            pl.BlockSpec((1, _C), lambda p, i: (0, 0)),
            pl.BlockSpec((1, _C), lambda p, i: (0, 0)),
        ],
        out_specs=pl.BlockSpec((_BLK, _C), lambda p, i: (p * i, 0)),
        out_shape=jax.ShapeDtypeStruct((_N, _C), jnp.float32),
        scratch_shapes=[
            pltpu.VMEM((_NI, _C), jnp.float32),
            pltpu.VMEM((_NI, _C), jnp.float32),
            pltpu.VMEM((_NI, 1), jnp.float32),
            pltpu.VMEM((_NI, 3 * _C), jnp.bfloat16),
            pltpu.VMEM((_N, _C), jnp.float32),
        ],
    )(seg_row, seg_col, features, w2, b2)
    return out


# speedup vs baseline: 10.4275x; 1.0339x over previous
import jax
import jax.numpy as jnp
from jax import lax
from jax.experimental import pallas as pl
from jax.experimental.pallas import tpu as pltpu

_N = 32768
_C = 256
_G = 32
_CPG = _C // _G
_NI = 64
_EPS = 1e-5
_BLK = 4096
_NBLK = _N // _BLK
_HI = lax.Precision.HIGHEST


def _hilo(v):
    hi = v.astype(jnp.bfloat16)
    lo = (v - hi.astype(jnp.float32)).astype(jnp.bfloat16)
    return hi, lo


def _body(seg_row_ref, seg_col_ref, x_ref, w_ref, b_ref, o_ref,
          s1_acc, s2_acc, cnt_acc, tab, xsave):
    p = pl.program_id(0)
    i = pl.program_id(1)

    @pl.when(p == 0)
    def _stats():
        @pl.when(i == 0)
        def _init():
            s1_acc[...] = jnp.zeros_like(s1_acc)
            s2_acc[...] = jnp.zeros_like(s2_acc)
            cnt_acc[...] = jnp.zeros_like(cnt_acc)

        x = x_ref[...]
        xsave[pl.ds(i * _BLK, _BLK), :] = x
        xb = x.astype(jnp.bfloat16)
        seg = seg_row_ref[0]
        onehot = (
            lax.broadcasted_iota(jnp.int32, (_NI, _BLK), 0) == seg
        ).astype(jnp.bfloat16)
        s1_acc[...] += jnp.dot(onehot, xb, preferred_element_type=jnp.float32)
        s2_acc[...] += jnp.dot(onehot, xb * xb, preferred_element_type=jnp.float32)
        cnt_acc[...] += jnp.sum(
            onehot.astype(jnp.float32), axis=1, keepdims=True
        )

        @pl.when(i == _NBLK - 1)
        def _epilogue():
            s12 = jnp.concatenate([s1_acc[...], s2_acc[...]], axis=1)
            eg = (
                lax.broadcasted_iota(jnp.int32, (2 * _C, 2 * _G), 0) // _CPG
                == lax.broadcasted_iota(jnp.int32, (2 * _C, 2 * _G), 1)
            ).astype(jnp.float32)
            sg = jnp.dot(s12, eg, precision=_HI)
            cnt = jnp.maximum(cnt_acc[...] * float(_CPG), 1.0)
            mean = sg[:, :_G] / cnt
            var = sg[:, _G:] / cnt - mean * mean
            inv = lax.rsqrt(var + _EPS)
            rg = (
                lax.broadcasted_iota(jnp.int32, (_G, _C), 0)
                == lax.broadcasted_iota(jnp.int32, (_G, _C), 1) // _CPG
            ).astype(jnp.float32)
            inv_c = jnp.dot(inv, rg, precision=_HI)
            mean_c = jnp.dot(mean, rg, precision=_HI)
            scale = inv_c * w_ref[...]
            shift = b_ref[...] - mean_c * scale
            tab[...] = jnp.concatenate(
                [scale.astype(jnp.bfloat16), shift.astype(jnp.bfloat16)], axis=1
            )

    @pl.when(p == 1)
    def _norm():
        segc = seg_col_ref[0]
        onehot = (
            segc == lax.broadcasted_iota(jnp.int32, (_BLK, _NI), 1)
        ).astype(jnp.bfloat16)
        t = tab[...]
        sc_tok = jnp.dot(onehot, t[:, :_C], preferred_element_type=jnp.float32)
        sh_tok = jnp.dot(onehot, t[:, _C:], preferred_element_type=jnp.float32)
        o_ref[...] = xsave[pl.ds(i * _BLK, _BLK), :] * sc_tok + sh_tok


def kernel(features, ins_indices_batch, ins_ids, weight, bias):
    del ins_ids
    seg = ins_indices_batch.astype(jnp.int32)
    seg_row = seg.reshape(_NBLK, 1, _BLK)
    seg_col = seg.reshape(_NBLK, _BLK, 1)
    w2 = weight.reshape(1, _C)
    b2 = bias.reshape(1, _C)

    out = pl.pallas_call(
        _body,
        grid=(2, _NBLK),
        in_specs=[
            pl.BlockSpec((1, 1, _BLK), lambda p, i: (i, 0, 0)),
            pl.BlockSpec((1, _BLK, 1), lambda p, i: (i, 0, 0)),
            pl.BlockSpec((_BLK, _C), lambda p, i: ((1 - p) * i, 0)),
            pl.BlockSpec((1, _C), lambda p, i: (0, 0)),
            pl.BlockSpec((1, _C), lambda p, i: (0, 0)),
        ],
        out_specs=pl.BlockSpec((_BLK, _C), lambda p, i: (p * i, 0)),
        out_shape=jax.ShapeDtypeStruct((_N, _C), jnp.float32),
        scratch_shapes=[
            pltpu.VMEM((_NI, _C), jnp.float32),
            pltpu.VMEM((_NI, _C), jnp.float32),
            pltpu.VMEM((_NI, 1), jnp.float32),
            pltpu.VMEM((_NI, 2 * _C), jnp.bfloat16),
            pltpu.VMEM((_N, _C), jnp.float32),
        ],
    )(seg_row, seg_col, features, w2, b2)
    return out
